# baseline (speedup 1.0000x reference)
import functools

import jax
import jax.numpy as jnp
from jax import lax
from jax.experimental import pallas as pl
from jax.experimental.pallas import tpu as pltpu
from jax.experimental.pallas import tpu_sc as plsc

N = 32768
D = 512
S = 256

L = 16
G = D // L
C = 16
NB = 8
NW = 32
SEG_PER_W = S // NW


def _sc_body(x_hbm, graph_hbm, out_hbm, graph_v, b0, b1, b2, b3, b4, b5, b6, b7, out_v,
             s0, s1, s2, s3, s4, s5, s6, s7):
    bufs = [b0, b1, b2, b3, b4, b5, b6, b7]
    sems = [s0, s1, s2, s3, s4, s5, s6, s7]

    cid = lax.axis_index("c")
    sid = lax.axis_index("s")
    wid = sid * 2 + cid

    pltpu.sync_copy(graph_hbm, graph_v)
    base_seg = wid * SEG_PER_W
    seg_end = base_seg + SEG_PER_W

    def mk_state(seg):
        segc = jnp.minimum(seg, S - 1)
        se = graph_v[pl.ds(segc * 2, L)]
        s = se[0]
        e = se[1]
        astart = s - lax.rem(s, 8)
        nch = lax.div(e + 1 - astart + (C - 1), C)
        return (seg, jnp.int32(0), s, e, astart, nch)

    def advance(st):
        seg, k, s, e, astart, nch = st
        last = (k + 1) >= nch
        nxt = mk_state(seg + 1)
        cur = (seg, k + 1, s, e, astart, nch)
        return tuple(jnp.where(last, a, b) for a, b in zip(nxt, cur))

    def start_c_of(st):
        seg, k, s, e, astart, nch = st
        return pl.multiple_of(jnp.minimum(astart + k * C, N - C), 8)

    def issue(st, buf, sem):
        pltpu.make_async_copy(
            x_hbm.at[pl.ds(start_c_of(st), C)], buf, sem
        ).start()

    def consume(st, buf, sem, acc):
        seg, k, s, e, astart, nch = st
        valid = seg < seg_end
        start = astart + k * C
        start_c = start_c_of(st)
        pltpu.make_async_copy(x_hbm.at[pl.ds(start_c, C)], buf, sem).wait()
        lo = jnp.maximum(s, start) - start_c
        hi = jnp.where(valid, jnp.minimum(e + 1, start + C) - start_c, lo)

        def row_body(r, a):
            return tuple(a[g] + buf[r, pl.ds(g * L, L)] for g in range(G))

        acc = lax.fori_loop(lo, hi, row_body, acc)

        last = ((k + 1) >= nch) & valid
        si = seg - base_seg

        @pl.when(last)
        def _():
            cnt_v = jnp.full((L,), e - s + 1, jnp.int32).astype(jnp.float32)
            inv = jnp.full((L,), 1.0, jnp.float32) / cnt_v
            for g in range(G):
                out_v[si, pl.ds(g * L, L)] = acc[g] * inv + 0.006

        zero = jnp.zeros((L,), jnp.float32)
        return tuple(jnp.where(last, zero, a) for a in acc)

    acc0 = tuple(jnp.zeros((L,), jnp.float32) for _ in range(G))

    total = jnp.int32(0)
    for i in range(SEG_PER_W):
        total = total + mk_state(base_seg + i)[5]
    n_iters = lax.div(total + (NB - 1), NB)

    st_a = mk_state(base_seg)
    for j in range(NB):
        issue(st_a, bufs[j], sems[j])
        st_a = advance(st_a)

    def body(_, carry):
        st_c, st_a, acc = carry
        for j in range(NB):
            acc = consume(st_c, bufs[j], sems[j], acc)
            issue(st_a, bufs[j], sems[j])
            st_c = advance(st_c)
            st_a = advance(st_a)
        return (st_c, st_a, acc)

    lax.fori_loop(0, n_iters, body, (mk_state(base_seg), st_a, acc0))

    for j in range(NB):
        pltpu.make_async_copy(x_hbm.at[pl.ds(0, C)], bufs[j], sems[j]).wait()

    pltpu.sync_copy(
        out_v.at[pl.ds(0, SEG_PER_W)], out_hbm.at[pl.ds(base_seg, SEG_PER_W)]
    )


@jax.jit
def _wise_pooling(x, graph):
    mesh = plsc.VectorSubcoreMesh(core_axis_name="c", subcore_axis_name="s")
    f = pl.kernel(
        _sc_body,
        out_type=jax.ShapeDtypeStruct((S, D), jnp.float32),
        mesh=mesh,
        scratch_types=[
            pltpu.VMEM((S * 2 + L,), jnp.int32),
            pltpu.VMEM((C, D), jnp.float32),
            pltpu.VMEM((C, D), jnp.float32),
            pltpu.VMEM((C, D), jnp.float32),
            pltpu.VMEM((C, D), jnp.float32),
            pltpu.VMEM((C, D), jnp.float32),
            pltpu.VMEM((C, D), jnp.float32),
            pltpu.VMEM((C, D), jnp.float32),
            pltpu.VMEM((C, D), jnp.float32),
            pltpu.VMEM((SEG_PER_W + 1, D), jnp.float32),
            pltpu.SemaphoreType.DMA,
            pltpu.SemaphoreType.DMA,
            pltpu.SemaphoreType.DMA,
            pltpu.SemaphoreType.DMA,
            pltpu.SemaphoreType.DMA,
            pltpu.SemaphoreType.DMA,
            pltpu.SemaphoreType.DMA,
            pltpu.SemaphoreType.DMA,
        ],
    )
    return f(x, graph)


def kernel(input, graph):
    gflat = jnp.pad(graph.astype(jnp.int32).reshape(-1), (0, L))
    return _wise_pooling(input, gflat)


# speedup vs baseline: 1.0303x; 1.0303x over previous
import functools

import jax
import jax.numpy as jnp
from jax import lax
from jax.experimental import pallas as pl
from jax.experimental.pallas import tpu as pltpu
from jax.experimental.pallas import tpu_sc as plsc

N = 32768
D = 512
S = 256

L = 16
G = D // L
C = 32
NB = 6
NW = 32
SEG_PER_W = S // NW


def _sc_body(x_hbm, graph_hbm, out_hbm, graph_v, b0, b1, b2, b3, b4, b5, out_v,
             s0, s1, s2, s3, s4, s5):
    bufs = [b0, b1, b2, b3, b4, b5]
    sems = [s0, s1, s2, s3, s4, s5]

    cid = lax.axis_index("c")
    sid = lax.axis_index("s")
    wid = sid * 2 + cid

    pltpu.sync_copy(graph_hbm, graph_v)
    base_seg = wid * SEG_PER_W
    seg_end = base_seg + SEG_PER_W

    def mk_state(seg):
        segc = jnp.minimum(seg, S - 1)
        se = graph_v[pl.ds(segc * 2, L)]
        s = se[0]
        e = se[1]
        astart = s - lax.rem(s, 8)
        nch = lax.div(e + 1 - astart + (C - 1), C)
        return (seg, jnp.int32(0), s, e, astart, nch)

    def advance(st):
        seg, k, s, e, astart, nch = st
        last = (k + 1) >= nch
        nxt = mk_state(seg + 1)
        cur = (seg, k + 1, s, e, astart, nch)
        return tuple(jnp.where(last, a, b) for a, b in zip(nxt, cur))

    def start_c_of(st):
        seg, k, s, e, astart, nch = st
        return pl.multiple_of(jnp.minimum(astart + k * C, N - C), 8)

    def issue(st, buf, sem):
        pltpu.make_async_copy(
            x_hbm.at[pl.ds(start_c_of(st), C)], buf, sem
        ).start()

    def consume(st, buf, sem, acc):
        seg, k, s, e, astart, nch = st
        valid = seg < seg_end
        start = astart + k * C
        start_c = start_c_of(st)
        pltpu.make_async_copy(x_hbm.at[pl.ds(start_c, C)], buf, sem).wait()
        lo = jnp.maximum(s, start) - start_c
        hi = jnp.where(valid, jnp.minimum(e + 1, start + C) - start_c, lo)

        def row_body(r, a):
            return tuple(a[g] + buf[r, pl.ds(g * L, L)] for g in range(G))

        acc = lax.fori_loop(lo, hi, row_body, acc)

        last = ((k + 1) >= nch) & valid
        si = seg - base_seg

        @pl.when(last)
        def _():
            cnt_v = jnp.full((L,), e - s + 1, jnp.int32).astype(jnp.float32)
            inv = jnp.full((L,), 1.0, jnp.float32) / cnt_v
            for g in range(G):
                out_v[si, pl.ds(g * L, L)] = acc[g] * inv + 0.006

        zero = jnp.zeros((L,), jnp.float32)
        return tuple(jnp.where(last, zero, a) for a in acc)

    acc0 = tuple(jnp.zeros((L,), jnp.float32) for _ in range(G))

    total = jnp.int32(0)
    for i in range(SEG_PER_W):
        total = total + mk_state(base_seg + i)[5]
    n_iters = lax.div(total + (NB - 1), NB)

    st_a = mk_state(base_seg)
    for j in range(NB):
        issue(st_a, bufs[j], sems[j])
        st_a = advance(st_a)

    def body(_, carry):
        st_c, st_a, acc = carry
        for j in range(NB):
            acc = consume(st_c, bufs[j], sems[j], acc)
            issue(st_a, bufs[j], sems[j])
            st_c = advance(st_c)
            st_a = advance(st_a)
        return (st_c, st_a, acc)

    lax.fori_loop(0, n_iters, body, (mk_state(base_seg), st_a, acc0))

    for j in range(NB):
        pltpu.make_async_copy(x_hbm.at[pl.ds(0, C)], bufs[j], sems[j]).wait()

    pltpu.sync_copy(
        out_v.at[pl.ds(0, SEG_PER_W)], out_hbm.at[pl.ds(base_seg, SEG_PER_W)]
    )


@jax.jit
def _wise_pooling(x, graph):
    mesh = plsc.VectorSubcoreMesh(core_axis_name="c", subcore_axis_name="s")
    f = pl.kernel(
        _sc_body,
        out_type=jax.ShapeDtypeStruct((S, D), jnp.float32),
        mesh=mesh,
        scratch_types=[
            pltpu.VMEM((S * 2 + L,), jnp.int32),
            pltpu.VMEM((C, D), jnp.float32),
            pltpu.VMEM((C, D), jnp.float32),
            pltpu.VMEM((C, D), jnp.float32),
            pltpu.VMEM((C, D), jnp.float32),
            pltpu.VMEM((C, D), jnp.float32),
            pltpu.VMEM((C, D), jnp.float32),
            pltpu.VMEM((SEG_PER_W + 1, D), jnp.float32),
            pltpu.SemaphoreType.DMA,
            pltpu.SemaphoreType.DMA,
            pltpu.SemaphoreType.DMA,
            pltpu.SemaphoreType.DMA,
            pltpu.SemaphoreType.DMA,
            pltpu.SemaphoreType.DMA,
        ],
    )
    return f(x, graph)


def kernel(input, graph):
    gflat = jnp.pad(graph.astype(jnp.int32).reshape(-1), (0, L))
    return _wise_pooling(input, gflat)


# speedup vs baseline: 1.0885x; 1.0565x over previous
import functools

import jax
import jax.numpy as jnp
from jax import lax
from jax.experimental import pallas as pl
from jax.experimental.pallas import tpu as pltpu
from jax.experimental.pallas import tpu_sc as plsc

N = 32768
D = 512
S = 256

L = 16
G = D // L
C = 32
NB = 4
NW = 32
SEG_PER_W = S // NW


def _sc_body(x_hbm, graph_hbm, out_hbm, graph_v, b0, b1, b2, b3, out_v,
             s0, s1, s2, s3):
    bufs = [b0, b1, b2, b3]
    sems = [s0, s1, s2, s3]

    cid = lax.axis_index("c")
    sid = lax.axis_index("s")
    wid = sid * 2 + cid

    pltpu.sync_copy(graph_hbm, graph_v)
    base_seg = wid * SEG_PER_W
    seg_end = base_seg + SEG_PER_W

    def mk_state(seg):
        segc = jnp.minimum(seg, S - 1)
        se = graph_v[pl.ds(segc * 2, L)]
        s = se[0]
        e = se[1]
        astart = s - lax.rem(s, 8)
        nch = lax.div(e + 1 - astart + (C - 1), C)
        return (seg, jnp.int32(0), s, e, astart, nch)

    def advance(st):
        seg, k, s, e, astart, nch = st
        last = (k + 1) >= nch
        nxt = mk_state(seg + 1)
        cur = (seg, k + 1, s, e, astart, nch)
        return tuple(jnp.where(last, a, b) for a, b in zip(nxt, cur))

    def start_c_of(st):
        seg, k, s, e, astart, nch = st
        return pl.multiple_of(jnp.minimum(astart + k * C, N - C), 8)

    def issue(st, buf, sem):
        pltpu.make_async_copy(
            x_hbm.at[pl.ds(start_c_of(st), C)], buf, sem
        ).start()

    def consume(st, buf, sem, acc):
        seg, k, s, e, astart, nch = st
        valid = seg < seg_end
        start = astart + k * C
        start_c = start_c_of(st)
        pltpu.make_async_copy(x_hbm.at[pl.ds(start_c, C)], buf, sem).wait()
        lo = jnp.maximum(s, start) - start_c
        hi = jnp.where(valid, jnp.minimum(e + 1, start + C) - start_c, lo)

        def row_body(r, a):
            return tuple(a[g] + buf[r, pl.ds(g * L, L)] for g in range(G))

        acc = lax.fori_loop(lo, hi, row_body, acc)

        last = ((k + 1) >= nch) & valid
        si = seg - base_seg

        @pl.when(last)
        def _():
            cnt_v = jnp.full((L,), e - s + 1, jnp.int32).astype(jnp.float32)
            inv = jnp.full((L,), 1.0, jnp.float32) / cnt_v
            for g in range(G):
                out_v[si, pl.ds(g * L, L)] = acc[g] * inv + 0.006

        zero = jnp.zeros((L,), jnp.float32)
        return tuple(jnp.where(last, zero, a) for a in acc)

    acc0 = tuple(jnp.zeros((L,), jnp.float32) for _ in range(G))

    total = jnp.int32(0)
    for i in range(SEG_PER_W):
        total = total + mk_state(base_seg + i)[5]
    n_iters = lax.div(total + (NB - 1), NB)

    st_a = mk_state(base_seg)
    for j in range(NB):
        issue(st_a, bufs[j], sems[j])
        st_a = advance(st_a)

    def body(_, carry):
        st_c, st_a, acc = carry
        for j in range(NB):
            acc = consume(st_c, bufs[j], sems[j], acc)
            issue(st_a, bufs[j], sems[j])
            st_c = advance(st_c)
            st_a = advance(st_a)
        return (st_c, st_a, acc)

    lax.fori_loop(0, n_iters, body, (mk_state(base_seg), st_a, acc0))

    for j in range(NB):
        pltpu.make_async_copy(x_hbm.at[pl.ds(0, C)], bufs[j], sems[j]).wait()

    pltpu.sync_copy(
        out_v.at[pl.ds(0, SEG_PER_W)], out_hbm.at[pl.ds(base_seg, SEG_PER_W)]
    )


@jax.jit
def _wise_pooling(x, graph):
    mesh = plsc.VectorSubcoreMesh(core_axis_name="c", subcore_axis_name="s")
    f = pl.kernel(
        _sc_body,
        out_type=jax.ShapeDtypeStruct((S, D), jnp.float32),
        mesh=mesh,
        scratch_types=[
            pltpu.VMEM((S * 2 + L,), jnp.int32),
            pltpu.VMEM((C, D), jnp.float32),
            pltpu.VMEM((C, D), jnp.float32),
            pltpu.VMEM((C, D), jnp.float32),
            pltpu.VMEM((C, D), jnp.float32),
            pltpu.VMEM((SEG_PER_W + 1, D), jnp.float32),
            pltpu.SemaphoreType.DMA,
            pltpu.SemaphoreType.DMA,
            pltpu.SemaphoreType.DMA,
            pltpu.SemaphoreType.DMA,
        ],
        compiler_params=pltpu.CompilerParams(
            disable_bounds_checks=True, disable_semaphore_checks=True
        ),
    )
    return f(x, graph)


def kernel(input, graph):
    gflat = jnp.pad(graph.astype(jnp.int32).reshape(-1), (0, L))
    return _wise_pooling(input, gflat)
